# Initial kernel scaffold; baseline (speedup 1.0000x reference)
import jax
import jax.numpy as jnp
from jax.experimental import pallas as pl


def kernel(target_items_context, interacted_items_context, user_embeds, attention_mask, w_dense, b_dense, W_mlp, b_mlp):
    raise NotImplementedError("write your pallas kernel here")


# speedup vs baseline: 2.8182x; 2.8182x over previous
import jax
import jax.numpy as jnp
from jax.experimental import pallas as pl
from jax.experimental.pallas import tpu as pltpu

_MASK_VALUE = -10000000.0


def _pooler_kernel(t_ref, k_ref, u_ref, mb_ref, wd_ref, bd_ref, wm_ref,
                   bm_ref, out_ref, attn_ref):
    C = t_ref.shape[2]
    E = u_ref.shape[2]
    t = t_ref[0]
    k = k_ref[0]
    w1 = wd_ref[:, :C]
    w2 = wd_ref[:, C:]
    s_t = jax.lax.dot_general(t, w1, (((1,), (1,)), ((), ())),
                              preferred_element_type=jnp.float32)
    s_k = jax.lax.dot_general(w2, k, (((1,), (1,)), ((), ())),
                              preferred_element_type=jnp.float32)
    scores = jnp.tanh(s_t + s_k + bd_ref[0, 0]) + mb_ref[0]
    m = jnp.max(scores, axis=1, keepdims=True)
    e = jnp.exp(scores - m)
    s = jnp.sum(e, axis=1, keepdims=True)
    attn = e / s
    attn_ref[0] = attn
    pooled = jnp.dot(attn, k, preferred_element_type=jnp.float32)
    u_part = jnp.dot(u_ref[0], wm_ref[:E, :],
                     preferred_element_type=jnp.float32)
    h = jnp.dot(pooled, wm_ref[E:, :], preferred_element_type=jnp.float32)
    out_ref[0] = jnp.maximum(h + u_part + bm_ref[:], 0.0)


def kernel(target_items_context, interacted_items_context, user_embeds,
           attention_mask, w_dense, b_dense, W_mlp, b_mlp):
    U, I, C = target_items_context.shape
    J = interacted_items_context.shape[1]
    E = user_embeds.shape[1]
    mask_bias = jnp.where(attention_mask, 0.0, _MASK_VALUE).astype(
        jnp.float32).reshape(U, 1, J)
    users3 = user_embeds.reshape(U, 1, E)
    wd = w_dense.reshape(1, 2 * C)
    bd = b_dense.reshape(1, 1)
    bm = b_mlp.reshape(1, C)
    out, attn = pl.pallas_call(
        _pooler_kernel,
        grid=(U,),
        in_specs=[
            pl.BlockSpec((1, I, C), lambda u: (u, 0, 0)),
            pl.BlockSpec((1, J, C), lambda u: (u, 0, 0)),
            pl.BlockSpec((1, 1, E), lambda u: (u, 0, 0)),
            pl.BlockSpec((1, 1, J), lambda u: (u, 0, 0)),
            pl.BlockSpec((1, 2 * C), lambda u: (0, 0)),
            pl.BlockSpec((1, 1), lambda u: (0, 0)),
            pl.BlockSpec((E + C, C), lambda u: (0, 0)),
            pl.BlockSpec((1, C), lambda u: (0, 0)),
        ],
        out_specs=[
            pl.BlockSpec((1, I, C), lambda u: (u, 0, 0)),
            pl.BlockSpec((1, I, J), lambda u: (u, 0, 0)),
        ],
        out_shape=[
            jax.ShapeDtypeStruct((U, I, C), jnp.float32),
            jax.ShapeDtypeStruct((U, I, J), jnp.float32),
        ],
        compiler_params=pltpu.CompilerParams(
            dimension_semantics=("parallel",),
        ),
    )(target_items_context, interacted_items_context, users3,
      mask_bias, wd, bd, W_mlp, bm)
    return out, attn


# speedup vs baseline: 4.1353x; 1.4674x over previous
import jax
import jax.numpy as jnp
from jax.experimental import pallas as pl
from jax.experimental.pallas import tpu as pltpu

_MASK_VALUE = -10000000.0


def _pooler_kernel(t_ref, k_ref, u_ref, mb_ref, wd_ref, bd_ref, wm_ref,
                   bm_ref, out_ref, attn_ref):
    BU = t_ref.shape[0]
    C = t_ref.shape[2]
    E = u_ref.shape[2]
    w1 = wd_ref[:, :C]
    w2 = wd_ref[:, C:]
    b = bd_ref[0, 0]
    for u in range(BU):
        t = t_ref[u]
        k = k_ref[u]
        s_t = jax.lax.dot_general(t, w1, (((1,), (1,)), ((), ())),
                                  preferred_element_type=jnp.float32)
        s_k = jax.lax.dot_general(w2, k, (((1,), (1,)), ((), ())),
                                  preferred_element_type=jnp.float32)
        scores = jnp.tanh(s_t + s_k + b) + mb_ref[u]
        m = jnp.max(scores, axis=1, keepdims=True)
        e = jnp.exp(scores - m)
        s = jnp.sum(e, axis=1, keepdims=True)
        attn = e / s
        attn_ref[u] = attn
        pooled = jnp.dot(attn, k, preferred_element_type=jnp.float32)
        u_part = jnp.dot(u_ref[u], wm_ref[:E, :],
                         preferred_element_type=jnp.float32)
        h = jnp.dot(pooled, wm_ref[E:, :],
                    preferred_element_type=jnp.float32)
        out_ref[u] = jnp.maximum(h + u_part + bm_ref[:], 0.0)


def kernel(target_items_context, interacted_items_context, user_embeds,
           attention_mask, w_dense, b_dense, W_mlp, b_mlp):
    U, I, C = target_items_context.shape
    J = interacted_items_context.shape[1]
    E = user_embeds.shape[1]
    BU = 4
    mask_bias = jnp.where(attention_mask, 0.0, _MASK_VALUE).astype(
        jnp.float32).reshape(U, 1, J)
    users3 = user_embeds.reshape(U, 1, E)
    wd = w_dense.reshape(1, 2 * C)
    bd = b_dense.reshape(1, 1)
    bm = b_mlp.reshape(1, C)
    out, attn = pl.pallas_call(
        _pooler_kernel,
        grid=(U // BU,),
        in_specs=[
            pl.BlockSpec((BU, I, C), lambda u: (u, 0, 0)),
            pl.BlockSpec((BU, J, C), lambda u: (u, 0, 0)),
            pl.BlockSpec((BU, 1, E), lambda u: (u, 0, 0)),
            pl.BlockSpec((BU, 1, J), lambda u: (u, 0, 0)),
            pl.BlockSpec((1, 2 * C), lambda u: (0, 0)),
            pl.BlockSpec((1, 1), lambda u: (0, 0)),
            pl.BlockSpec((E + C, C), lambda u: (0, 0)),
            pl.BlockSpec((1, C), lambda u: (0, 0)),
        ],
        out_specs=[
            pl.BlockSpec((BU, I, C), lambda u: (u, 0, 0)),
            pl.BlockSpec((BU, I, J), lambda u: (u, 0, 0)),
        ],
        out_shape=[
            jax.ShapeDtypeStruct((U, I, C), jnp.float32),
            jax.ShapeDtypeStruct((U, I, J), jnp.float32),
        ],
        compiler_params=pltpu.CompilerParams(
            dimension_semantics=("parallel",),
        ),
    )(target_items_context, interacted_items_context, users3,
      mask_bias, wd, bd, W_mlp, bm)
    return out, attn


# speedup vs baseline: 4.2325x; 1.0235x over previous
import jax
import jax.numpy as jnp
from jax.experimental import pallas as pl
from jax.experimental.pallas import tpu as pltpu

_MASK_VALUE = -10000000.0


def _pooler_kernel(t_ref, k_ref, u_ref, mb_ref, wd_ref, bd_ref, wm_ref,
                   bm_ref, out_ref, attn_ref):
    BU = t_ref.shape[0]
    C = t_ref.shape[2]
    E = u_ref.shape[2]
    w1 = wd_ref[:, :C]
    w2 = wd_ref[:, C:]
    b = bd_ref[0, 0]
    for u in range(BU):
        t = t_ref[u]
        k = k_ref[u]
        s_t = jax.lax.dot_general(t, w1, (((1,), (1,)), ((), ())),
                                  preferred_element_type=jnp.float32)
        s_k = jax.lax.dot_general(w2, k, (((1,), (1,)), ((), ())),
                                  preferred_element_type=jnp.float32)
        scores = jnp.tanh(s_t + s_k + b) + mb_ref[u]
        m = jnp.max(scores, axis=1, keepdims=True)
        e = jnp.exp(scores - m)
        s = jnp.sum(e, axis=1, keepdims=True)
        attn = e / s
        attn_ref[u] = attn
        pooled = jnp.dot(attn, k, preferred_element_type=jnp.float32)
        u_part = jnp.dot(u_ref[u], wm_ref[:E, :],
                         preferred_element_type=jnp.float32)
        h = jnp.dot(pooled, wm_ref[E:, :],
                    preferred_element_type=jnp.float32)
        out_ref[u] = jnp.maximum(h + u_part + bm_ref[:], 0.0)


def kernel(target_items_context, interacted_items_context, user_embeds,
           attention_mask, w_dense, b_dense, W_mlp, b_mlp):
    U, I, C = target_items_context.shape
    J = interacted_items_context.shape[1]
    E = user_embeds.shape[1]
    BU = 8
    mask_bias = jnp.where(attention_mask, 0.0, _MASK_VALUE).astype(
        jnp.float32).reshape(U, 1, J)
    users3 = user_embeds.reshape(U, 1, E)
    wd = w_dense.reshape(1, 2 * C)
    bd = b_dense.reshape(1, 1)
    bm = b_mlp.reshape(1, C)
    out, attn = pl.pallas_call(
        _pooler_kernel,
        grid=(U // BU,),
        in_specs=[
            pl.BlockSpec((BU, I, C), lambda u: (u, 0, 0)),
            pl.BlockSpec((BU, J, C), lambda u: (u, 0, 0)),
            pl.BlockSpec((BU, 1, E), lambda u: (u, 0, 0)),
            pl.BlockSpec((BU, 1, J), lambda u: (u, 0, 0)),
            pl.BlockSpec((1, 2 * C), lambda u: (0, 0)),
            pl.BlockSpec((1, 1), lambda u: (0, 0)),
            pl.BlockSpec((E + C, C), lambda u: (0, 0)),
            pl.BlockSpec((1, C), lambda u: (0, 0)),
        ],
        out_specs=[
            pl.BlockSpec((BU, I, C), lambda u: (u, 0, 0)),
            pl.BlockSpec((BU, I, J), lambda u: (u, 0, 0)),
        ],
        out_shape=[
            jax.ShapeDtypeStruct((U, I, C), jnp.float32),
            jax.ShapeDtypeStruct((U, I, J), jnp.float32),
        ],
        compiler_params=pltpu.CompilerParams(
            dimension_semantics=("parallel",),
        ),
    )(target_items_context, interacted_items_context, users3,
      mask_bias, wd, bd, W_mlp, bm)
    return out, attn


# speedup vs baseline: 4.4805x; 1.0586x over previous
import jax
import jax.numpy as jnp
from jax.experimental import pallas as pl
from jax.experimental.pallas import tpu as pltpu

_MASK_VALUE = -10000000.0


def _pooler_kernel(t_ref, k_ref, u_ref, mb_ref, wd_ref, bd_ref, wm_ref,
                   bm_ref, out_ref, attn_ref):
    BU = t_ref.shape[0]
    C = t_ref.shape[2]
    E = u_ref.shape[2]
    w1 = wd_ref[:, :C]
    w2 = wd_ref[:, C:]
    b = bd_ref[0, 0]
    for u in range(BU):
        t = t_ref[u]
        k = k_ref[u]
        s_t = jax.lax.dot_general(t, w1, (((1,), (1,)), ((), ())),
                                  preferred_element_type=jnp.float32)
        s_k = jax.lax.dot_general(w2, k, (((1,), (1,)), ((), ())),
                                  preferred_element_type=jnp.float32)
        e = jnp.exp(jnp.tanh(s_t + (s_k + b))) * mb_ref[u]
        s = jnp.sum(e, axis=1, keepdims=True)
        attn = e / s
        attn_ref[u] = attn
        pooled = jnp.dot(attn, k, preferred_element_type=jnp.float32)
        u_part = jnp.dot(u_ref[u], wm_ref[:E, :],
                         preferred_element_type=jnp.float32)
        h = jnp.dot(pooled, wm_ref[E:, :],
                    preferred_element_type=jnp.float32)
        out_ref[u] = jnp.maximum(h + u_part + bm_ref[:], 0.0)


def kernel(target_items_context, interacted_items_context, user_embeds,
           attention_mask, w_dense, b_dense, W_mlp, b_mlp):
    U, I, C = target_items_context.shape
    J = interacted_items_context.shape[1]
    E = user_embeds.shape[1]
    BU = 8
    mask01 = attention_mask.astype(jnp.float32).reshape(U, 1, J)
    users3 = user_embeds.reshape(U, 1, E)
    wd = w_dense.reshape(1, 2 * C)
    bd = b_dense.reshape(1, 1)
    bm = b_mlp.reshape(1, C)
    out, attn = pl.pallas_call(
        _pooler_kernel,
        grid=(U // BU,),
        in_specs=[
            pl.BlockSpec((BU, I, C), lambda u: (u, 0, 0)),
            pl.BlockSpec((BU, J, C), lambda u: (u, 0, 0)),
            pl.BlockSpec((BU, 1, E), lambda u: (u, 0, 0)),
            pl.BlockSpec((BU, 1, J), lambda u: (u, 0, 0)),
            pl.BlockSpec((1, 2 * C), lambda u: (0, 0)),
            pl.BlockSpec((1, 1), lambda u: (0, 0)),
            pl.BlockSpec((E + C, C), lambda u: (0, 0)),
            pl.BlockSpec((1, C), lambda u: (0, 0)),
        ],
        out_specs=[
            pl.BlockSpec((BU, I, C), lambda u: (u, 0, 0)),
            pl.BlockSpec((BU, I, J), lambda u: (u, 0, 0)),
        ],
        out_shape=[
            jax.ShapeDtypeStruct((U, I, C), jnp.float32),
            jax.ShapeDtypeStruct((U, I, J), jnp.float32),
        ],
        compiler_params=pltpu.CompilerParams(
            dimension_semantics=("parallel",),
        ),
    )(target_items_context, interacted_items_context, users3,
      mask01, wd, bd, W_mlp, bm)
    return out, attn
